# baseline (speedup 1.0000x reference)
import functools

import jax
import jax.numpy as jnp
from jax import lax
from jax.experimental import pallas as pl
from jax.experimental.pallas import tpu as pltpu
from jax.experimental.pallas import tpu_sc as plsc

N = 10000
E = 160000
H = 256
NORM = 0.01

NPAD = 10240
CH = 128
NCHUNK = E // CH
NBLK = 1000
EBLK = 1280
NW_G = 32
NW_S = 16
NPW = NPAD // NW_S


def _precompute_body(h1_ref, h2_ref, w0a_ref, w0b_ref, a1_ref, a2_ref):
    a1_ref[...] = jnp.dot(h1_ref[...], w0a_ref[...],
                          preferred_element_type=jnp.float32
                          ).astype(jnp.bfloat16)
    a2_ref[...] = jnp.dot(h2_ref[...], w0b_ref[...],
                          preferred_element_type=jnp.float32
                          ).astype(jnp.bfloat16)


_precompute = pl.pallas_call(
    _precompute_body,
    grid=(N // NBLK,),
    in_specs=[
        pl.BlockSpec((NBLK, H), lambda i: (i, 0)),
        pl.BlockSpec((NBLK, H), lambda i: (i, 0)),
        pl.BlockSpec((H, H), lambda i: (0, 0)),
        pl.BlockSpec((H, H), lambda i: (0, 0)),
    ],
    out_specs=[
        pl.BlockSpec((NBLK, H), lambda i: (i, 0)),
        pl.BlockSpec((NBLK, H), lambda i: (i, 0)),
    ],
    out_shape=[
        jax.ShapeDtypeStruct((N, H), jnp.bfloat16),
        jax.ShapeDtypeStruct((N, H), jnp.bfloat16),
    ],
)


_MESH_G = plsc.VectorSubcoreMesh(core_axis_name="c", subcore_axis_name="s")


MAXG = 40


HW = H // 2


@functools.partial(
    pl.kernel,
    out_type=(jax.ShapeDtypeStruct((E, HW), jnp.int32),
              jax.ShapeDtypeStruct((E, HW), jnp.int32)),
    mesh=_MESH_G,
    scratch_types=[
        pltpu.VMEM((MAXG, CH), jnp.int32),
        pltpu.VMEM((MAXG, CH), jnp.int32),
        pltpu.VMEM((CH, HW), jnp.int32),
        pltpu.VMEM((CH, HW), jnp.int32),
        pltpu.VMEM((CH, HW), jnp.int32),
        pltpu.VMEM((CH, HW), jnp.int32),
        pltpu.SemaphoreType.DMA,
        pltpu.SemaphoreType.DMA,
    ],
)
def _gather_rows(n1r, n2r, a1, a2, g1, g2, idx1_2d, idx2_2d, r1a, r1b,
                 r2a, r2b, semL, semG):
    wid = lax.axis_index("s") * 2 + lax.axis_index("c")
    rows1 = (r1a, r1b)
    rows2 = (r2a, r2b)
    start = pl.multiple_of(wid * MAXG, 8)
    nchunks = jnp.minimum(MAXG, NCHUNK - wid * MAXG)
    cpi1 = pltpu.async_copy(n1r.at[pl.ds(start, MAXG)], idx1_2d, semL)
    cpi2 = pltpu.async_copy(n2r.at[pl.ds(start, MAXG)], idx2_2d, semL)
    cpi1.wait()
    cpi2.wait()

    def fire(j, s):
        pltpu.async_copy(a1.at[idx1_2d.at[j]], rows1[s], semG)
        pltpu.async_copy(a2.at[idx2_2d.at[j]], rows2[s], semG)

    def wait_pair(s):
        pltpu.make_async_copy(a1.at[pl.ds(0, CH)], rows1[s], semG).wait()
        pltpu.make_async_copy(a2.at[pl.ds(0, CH)], rows2[s], semG).wait()

    fire(0, 0)

    def body(ip, carry):
        for s in (0, 1):
            j = 2 * ip + s

            @pl.when(j + 1 < nchunks)
            def _():
                fire(j + 1, 1 - s)

            wait_pair(s)
            base = pl.multiple_of((wid * MAXG + j) * CH, CH)
            pltpu.sync_copy(rows1[s], g1.at[pl.ds(base, CH)])
            pltpu.sync_copy(rows2[s], g2.at[pl.ds(base, CH)])
        return carry

    lax.fori_loop(0, nchunks // 2, body, 0)


def _mlp_body(g1, g2, ea, cd, w0ce, w0co, b0e, b0o, w1e, w1o, b1, w2t,
              outx, outy, outz):
    i1 = g1[...]
    i2 = g2[...]
    hi = jnp.int32(-65536)
    e1 = lax.bitcast_convert_type(i1 << 16, jnp.float32)
    o1 = lax.bitcast_convert_type(i1 & hi, jnp.float32)
    e2 = lax.bitcast_convert_type(i2 << 16, jnp.float32)
    o2 = lax.bitcast_convert_type(i2 & hi, jnp.float32)
    eav = ea[...]
    pre_e = e1 + e2 + eav * w0ce[...] + b0e[...]
    pre_o = o1 + o2 + eav * w0co[...] + b0o[...]
    x0e = (pre_e * jax.nn.sigmoid(pre_e)).astype(jnp.bfloat16)
    x0o = (pre_o * jax.nn.sigmoid(pre_o)).astype(jnp.bfloat16)
    pre1 = (jnp.dot(x0e, w1e[...], preferred_element_type=jnp.float32)
            + jnp.dot(x0o, w1o[...], preferred_element_type=jnp.float32)
            + b1[...])
    x1 = pre1 * jax.nn.sigmoid(pre1)
    m = jnp.sum(x1 * w2t[...], axis=1, keepdims=True)
    trans = cd[...] * m
    outx[...] = trans[:, 0].reshape(1, EBLK // CH, CH)
    outy[...] = trans[:, 1].reshape(1, EBLK // CH, CH)
    outz[...] = trans[:, 2].reshape(1, EBLK // CH, CH)


_mlp = pl.pallas_call(
    _mlp_body,
    grid=(E // EBLK,),
    in_specs=[
        pl.BlockSpec((EBLK, HW), lambda i: (i, 0)),
        pl.BlockSpec((EBLK, HW), lambda i: (i, 0)),
        pl.BlockSpec((EBLK, 1), lambda i: (i, 0)),
        pl.BlockSpec((EBLK, 3), lambda i: (i, 0)),
        pl.BlockSpec((1, HW), lambda i: (0, 0)),
        pl.BlockSpec((1, HW), lambda i: (0, 0)),
        pl.BlockSpec((1, HW), lambda i: (0, 0)),
        pl.BlockSpec((1, HW), lambda i: (0, 0)),
        pl.BlockSpec((HW, H), lambda i: (0, 0)),
        pl.BlockSpec((HW, H), lambda i: (0, 0)),
        pl.BlockSpec((1, H), lambda i: (0, 0)),
        pl.BlockSpec((1, H), lambda i: (0, 0)),
    ],
    out_specs=[pl.BlockSpec((1, EBLK // CH, CH), lambda i: (i, 0, 0))] * 3,
    out_shape=[jax.ShapeDtypeStruct((E // EBLK, EBLK // CH, CH),
                                    jnp.float32)] * 3,
)


_MESH_S = plsc.VectorSubcoreMesh(core_axis_name="c", subcore_axis_name="s",
                                 num_cores=1)


MAXC = 80
NCHUNKP = MAXC * NW_S
DRAIN = 8


@functools.partial(
    pl.kernel,
    out_type=tuple(jax.ShapeDtypeStruct((NPAD,), jnp.float32)
                   for _ in range(3)),
    mesh=_MESH_S,
    scratch_types=[
        pltpu.VMEM((MAXC, CH), jnp.int32),
        pltpu.VMEM((MAXC * CH,), jnp.float32),
        pltpu.VMEM((MAXC * CH,), jnp.float32),
        pltpu.VMEM((MAXC * CH,), jnp.float32),
        pltpu.VMEM((NPW,), jnp.float32),
        pltpu.VMEM((CH * 3,), jnp.float32),
        pltpu.VMEM_SHARED((NPAD,), jnp.float32),
        pltpu.VMEM_SHARED((NPAD,), jnp.float32),
        pltpu.VMEM_SHARED((NPAD,), jnp.float32),
        pltpu.SemaphoreType.DMA,
        pltpu.SemaphoreType.DMA,
    ],
)
def _scatter_combine(n1r, tx, ty, tz, c1x, c1y, c1z, ox, oy, oz, idx2d,
                     stx, sty, stz, zbuf, dummy_v, acc_x, acc_y,
                     acc_z, semL, semS):
    sid = lax.axis_index("s")
    accs = (acc_x, acc_y, acc_z)
    stg = (stx, sty, stz)
    t_c = (tx, ty, tz)
    c1_c = (c1x, c1y, c1z)
    out_c = (ox, oy, oz)

    start = pl.multiple_of(sid * MAXC, 8)
    nchunks = jnp.minimum(MAXC, NCHUNK - sid * MAXC)
    cps = [pltpu.async_copy(n1r.at[pl.ds(start, MAXC)], idx2d, semL)]
    for comp in range(3):
        cps.append(pltpu.async_copy(
            t_c[comp].at[pl.ds(start * CH, MAXC * CH)], stg[comp], semL))

    def zbody(i, carry):
        zbuf[pl.ds(pl.multiple_of(i * 16, 16), 16)] = jnp.zeros(
            (16,), jnp.float32)
        return carry

    lax.fori_loop(0, NPW // 16, zbody, 0)
    for comp in range(3):
        pltpu.sync_copy(zbuf, accs[comp].at[pl.ds(sid * NPW, NPW)])
    plsc.subcore_barrier()
    for cp in cps:
        cp.wait()

    def fire(j, carry):
        sbase = pl.multiple_of(j * CH, CH)
        for comp in range(3):
            pltpu.async_copy(stg[comp].at[pl.ds(sbase, CH)],
                             accs[comp].at[idx2d.at[j]], semS, add=True)

        @pl.when(j >= DRAIN)
        def _():
            pltpu.make_async_copy(tx.at[pl.ds(0, CH * 3)], dummy_v,
                                  semS).wait()

        return carry

    lax.fori_loop(0, nchunks, fire, 0)

    def drain(j, carry):
        pltpu.make_async_copy(tx.at[pl.ds(0, CH * 3)], dummy_v,
                              semS).wait()
        return carry

    lax.fori_loop(0, DRAIN, drain, 0)
    plsc.subcore_barrier()

    for comp in range(3):
        pltpu.sync_copy(accs[comp].at[pl.ds(sid * NPW, NPW)],
                        stg[comp].at[pl.ds(0, NPW)])
        pltpu.sync_copy(c1_c[comp].at[pl.ds(sid * NPW, NPW)], zbuf)

        def fbody(i, carry):
            sl = pl.ds(pl.multiple_of(i * 16, 16), 16)
            stg[comp][sl] = zbuf[sl] + stg[comp][sl] * jnp.float32(NORM)
            return carry

        lax.fori_loop(0, NPW // 16, fbody, 0)
        pltpu.sync_copy(stg[comp].at[pl.ds(0, NPW)],
                        out_c[comp].at[pl.ds(sid * NPW, NPW)])


def kernel(h1, h2, coord1, coord2, edge_index, coord_diff, edge_attr, W0,
           b0, W1, b1, W2):
    del coord2
    n1 = edge_index[0].astype(jnp.int32)
    n2 = edge_index[1].astype(jnp.int32)
    w0a = W0[:H]
    w0b = W0[H:2 * H]
    w0c = W0[2 * H:].reshape(1, H)
    b0r = b0.reshape(1, H)
    b1r = b1.reshape(1, H)
    w2t = W2.reshape(1, H)

    n1r = jnp.pad(n1.reshape(NCHUNK, CH), ((0, NCHUNKP - NCHUNK), (0, 0)))
    n2r = jnp.pad(n2.reshape(NCHUNK, CH), ((0, NCHUNKP - NCHUNK), (0, 0)))
    a1, a2 = _precompute(h1, h2, w0a, w0b)
    a1i = lax.bitcast_convert_type(a1.reshape(N, HW, 2), jnp.int32)
    a2i = lax.bitcast_convert_type(a2.reshape(N, HW, 2), jnp.int32)
    g1, g2 = _gather_rows(n1r, n2r, a1i, a2i)
    w1b = W1.astype(jnp.bfloat16)
    txp, typ, tzp = _mlp(g1, g2, edge_attr, coord_diff,
                         w0c[:, 0::2], w0c[:, 1::2],
                         b0r[:, 0::2], b0r[:, 1::2],
                         w1b[0::2], w1b[1::2], b1r, w2t)
    c1p = jnp.pad(coord1, ((0, NPAD - N), (0, 0)))
    epad = (NCHUNKP - NCHUNK) * CH
    ox, oy, oz = _scatter_combine(
        n1r, jnp.pad(txp.reshape(E), (0, epad)),
        jnp.pad(typ.reshape(E), (0, epad)),
        jnp.pad(tzp.reshape(E), (0, epad)),
        c1p[:, 0], c1p[:, 1], c1p[:, 2])
    return jnp.stack([ox, oy, oz], axis=1)[:N]


# speedup vs baseline: 1.6760x; 1.6760x over previous
import functools

import jax
import jax.numpy as jnp
from jax import lax
from jax.experimental import pallas as pl
from jax.experimental.pallas import tpu as pltpu
from jax.experimental.pallas import tpu_sc as plsc

N = 10000
E = 160000
H = 256
NORM = 0.01

NPAD = 10240
CH = 128
NCHUNK = E // CH
NBLK = 1000
EBLK = 1280
NW_G = 32
NW_S = 16
NPW = NPAD // NW_S


HW = H // 2


def _pack_bf16_pair(even_f32, odd_f32):
    ie = lax.bitcast_convert_type(
        even_f32.astype(jnp.bfloat16).astype(jnp.float32), jnp.int32)
    io = lax.bitcast_convert_type(
        odd_f32.astype(jnp.bfloat16).astype(jnp.float32), jnp.int32)
    return io | lax.shift_right_logical(ie, 16)


def _precompute_body(h1_ref, h2_ref, w0ae_ref, w0ao_ref, w0be_ref,
                     w0bo_ref, a1_ref, a2_ref):
    a1_ref[...] = _pack_bf16_pair(
        jnp.dot(h1_ref[...], w0ae_ref[...],
                preferred_element_type=jnp.float32),
        jnp.dot(h1_ref[...], w0ao_ref[...],
                preferred_element_type=jnp.float32))
    a2_ref[...] = _pack_bf16_pair(
        jnp.dot(h2_ref[...], w0be_ref[...],
                preferred_element_type=jnp.float32),
        jnp.dot(h2_ref[...], w0bo_ref[...],
                preferred_element_type=jnp.float32))


_precompute = pl.pallas_call(
    _precompute_body,
    grid=(N // NBLK,),
    in_specs=[
        pl.BlockSpec((NBLK, H), lambda i: (i, 0)),
        pl.BlockSpec((NBLK, H), lambda i: (i, 0)),
        pl.BlockSpec((H, HW), lambda i: (0, 0)),
        pl.BlockSpec((H, HW), lambda i: (0, 0)),
        pl.BlockSpec((H, HW), lambda i: (0, 0)),
        pl.BlockSpec((H, HW), lambda i: (0, 0)),
    ],
    out_specs=[
        pl.BlockSpec((NBLK, HW), lambda i: (i, 0)),
        pl.BlockSpec((NBLK, HW), lambda i: (i, 0)),
    ],
    out_shape=[
        jax.ShapeDtypeStruct((N, HW), jnp.int32),
        jax.ShapeDtypeStruct((N, HW), jnp.int32),
    ],
)


_MESH_G = plsc.VectorSubcoreMesh(core_axis_name="c", subcore_axis_name="s")


MAXG = 40


@functools.partial(
    pl.kernel,
    out_type=(jax.ShapeDtypeStruct((E, HW), jnp.int32),
              jax.ShapeDtypeStruct((E, HW), jnp.int32)),
    mesh=_MESH_G,
    scratch_types=[
        pltpu.VMEM((MAXG, CH), jnp.int32),
        pltpu.VMEM((MAXG, CH), jnp.int32),
        pltpu.VMEM((CH, HW), jnp.int32),
        pltpu.VMEM((CH, HW), jnp.int32),
        pltpu.VMEM((CH, HW), jnp.int32),
        pltpu.VMEM((CH, HW), jnp.int32),
        pltpu.SemaphoreType.DMA,
        pltpu.SemaphoreType.DMA,
    ],
)
def _gather_rows(n1r, n2r, a1, a2, g1, g2, idx1_2d, idx2_2d, r1a, r1b,
                 r2a, r2b, semL, semG):
    wid = lax.axis_index("s") * 2 + lax.axis_index("c")
    rows1 = (r1a, r1b)
    rows2 = (r2a, r2b)
    start = pl.multiple_of(wid * MAXG, 8)
    nchunks = jnp.minimum(MAXG, NCHUNK - wid * MAXG)
    cpi1 = pltpu.async_copy(n1r.at[pl.ds(start, MAXG)], idx1_2d, semL)
    cpi2 = pltpu.async_copy(n2r.at[pl.ds(start, MAXG)], idx2_2d, semL)
    cpi1.wait()
    cpi2.wait()

    def fire(j, s):
        pltpu.async_copy(a1.at[idx1_2d.at[j]], rows1[s], semG)
        pltpu.async_copy(a2.at[idx2_2d.at[j]], rows2[s], semG)

    def wait_pair(s):
        pltpu.make_async_copy(a1.at[pl.ds(0, CH)], rows1[s], semG).wait()
        pltpu.make_async_copy(a2.at[pl.ds(0, CH)], rows2[s], semG).wait()

    fire(0, 0)

    def body(ip, carry):
        for s in (0, 1):
            j = 2 * ip + s

            @pl.when(j + 1 < nchunks)
            def _():
                fire(j + 1, 1 - s)

            wait_pair(s)
            base = pl.multiple_of((wid * MAXG + j) * CH, CH)
            pltpu.sync_copy(rows1[s], g1.at[pl.ds(base, CH)])
            pltpu.sync_copy(rows2[s], g2.at[pl.ds(base, CH)])
        return carry

    lax.fori_loop(0, nchunks // 2, body, 0)


def _mlp_body(g1, g2, ea, cd, w0ce, w0co, b0e, b0o, w1p, b1, w2t,
              outx, outy, outz):
    i1 = g1[...]
    i2 = g2[...]
    hi = jnp.int32(-65536)
    e1 = lax.bitcast_convert_type(i1 << 16, jnp.float32)
    o1 = lax.bitcast_convert_type(i1 & hi, jnp.float32)
    e2 = lax.bitcast_convert_type(i2 << 16, jnp.float32)
    o2 = lax.bitcast_convert_type(i2 & hi, jnp.float32)
    eav = ea[...]
    pre_e = e1 + e2 + eav * w0ce[...] + b0e[...]
    pre_o = o1 + o2 + eav * w0co[...] + b0o[...]
    x0e = (pre_e * jax.nn.sigmoid(pre_e)).astype(jnp.bfloat16)
    x0o = (pre_o * jax.nn.sigmoid(pre_o)).astype(jnp.bfloat16)
    x0c = jnp.concatenate([x0e, x0o], axis=1)
    pre1 = (jnp.dot(x0c, w1p[...], preferred_element_type=jnp.float32)
            + b1[...])
    x1 = pre1 * jax.nn.sigmoid(pre1)
    m = jnp.sum(x1 * w2t[...], axis=1, keepdims=True)
    trans = cd[...] * m
    outx[...] = trans[:, 0].reshape(1, EBLK // CH, CH)
    outy[...] = trans[:, 1].reshape(1, EBLK // CH, CH)
    outz[...] = trans[:, 2].reshape(1, EBLK // CH, CH)


_mlp = pl.pallas_call(
    _mlp_body,
    grid=(E // EBLK,),
    in_specs=[
        pl.BlockSpec((EBLK, HW), lambda i: (i, 0)),
        pl.BlockSpec((EBLK, HW), lambda i: (i, 0)),
        pl.BlockSpec((EBLK, 1), lambda i: (i, 0)),
        pl.BlockSpec((EBLK, 3), lambda i: (i, 0)),
        pl.BlockSpec((1, HW), lambda i: (0, 0)),
        pl.BlockSpec((1, HW), lambda i: (0, 0)),
        pl.BlockSpec((1, HW), lambda i: (0, 0)),
        pl.BlockSpec((1, HW), lambda i: (0, 0)),
        pl.BlockSpec((H, H), lambda i: (0, 0)),
        pl.BlockSpec((1, H), lambda i: (0, 0)),
        pl.BlockSpec((1, H), lambda i: (0, 0)),
    ],
    out_specs=[pl.BlockSpec((1, EBLK // CH, CH), lambda i: (i, 0, 0))] * 3,
    out_shape=[jax.ShapeDtypeStruct((E // EBLK, EBLK // CH, CH),
                                    jnp.float32)] * 3,
)


_MESH_S = plsc.VectorSubcoreMesh(core_axis_name="c", subcore_axis_name="s",
                                 num_cores=1)


MAXC = 80
NCHUNKP = MAXC * NW_S
DRAIN = 8


@functools.partial(
    pl.kernel,
    out_type=tuple(jax.ShapeDtypeStruct((NPAD,), jnp.float32)
                   for _ in range(3)),
    mesh=_MESH_S,
    scratch_types=[
        pltpu.VMEM((MAXC, CH), jnp.int32),
        pltpu.VMEM((MAXC * CH,), jnp.float32),
        pltpu.VMEM((MAXC * CH,), jnp.float32),
        pltpu.VMEM((MAXC * CH,), jnp.float32),
        pltpu.VMEM((NPW,), jnp.float32),
        pltpu.VMEM((CH * 3,), jnp.float32),
        pltpu.VMEM_SHARED((NPAD,), jnp.float32),
        pltpu.VMEM_SHARED((NPAD,), jnp.float32),
        pltpu.VMEM_SHARED((NPAD,), jnp.float32),
        pltpu.SemaphoreType.DMA,
        pltpu.SemaphoreType.DMA,
    ],
)
def _scatter_combine(n1r, tx, ty, tz, c1x, c1y, c1z, ox, oy, oz, idx2d,
                     stx, sty, stz, zbuf, dummy_v, acc_x, acc_y,
                     acc_z, semL, semS):
    sid = lax.axis_index("s")
    accs = (acc_x, acc_y, acc_z)
    stg = (stx, sty, stz)
    t_c = (tx, ty, tz)
    c1_c = (c1x, c1y, c1z)
    out_c = (ox, oy, oz)

    start = pl.multiple_of(sid * MAXC, 8)
    nchunks = jnp.minimum(MAXC, NCHUNK - sid * MAXC)
    cps = [pltpu.async_copy(n1r.at[pl.ds(start, MAXC)], idx2d, semL)]
    for comp in range(3):
        cps.append(pltpu.async_copy(
            t_c[comp].at[pl.ds(start * CH, MAXC * CH)], stg[comp], semL))

    def zbody(i, carry):
        zbuf[pl.ds(pl.multiple_of(i * 16, 16), 16)] = jnp.zeros(
            (16,), jnp.float32)
        return carry

    lax.fori_loop(0, NPW // 16, zbody, 0)
    for comp in range(3):
        pltpu.sync_copy(zbuf, accs[comp].at[pl.ds(sid * NPW, NPW)])
    plsc.subcore_barrier()
    for cp in cps:
        cp.wait()

    def fire(j, carry):
        sbase = pl.multiple_of(j * CH, CH)
        for comp in range(3):
            pltpu.async_copy(stg[comp].at[pl.ds(sbase, CH)],
                             accs[comp].at[idx2d.at[j]], semS, add=True)

        @pl.when(j >= DRAIN)
        def _():
            pltpu.make_async_copy(tx.at[pl.ds(0, CH * 3)], dummy_v,
                                  semS).wait()

        return carry

    lax.fori_loop(0, nchunks, fire, 0)

    def drain(j, carry):
        pltpu.make_async_copy(tx.at[pl.ds(0, CH * 3)], dummy_v,
                              semS).wait()
        return carry

    lax.fori_loop(0, DRAIN, drain, 0)
    plsc.subcore_barrier()

    for comp in range(3):
        pltpu.sync_copy(accs[comp].at[pl.ds(sid * NPW, NPW)],
                        stg[comp].at[pl.ds(0, NPW)])
        pltpu.sync_copy(c1_c[comp].at[pl.ds(sid * NPW, NPW)], zbuf)

        def fbody(i, carry):
            sl = pl.ds(pl.multiple_of(i * 16, 16), 16)
            stg[comp][sl] = zbuf[sl] + stg[comp][sl] * jnp.float32(NORM)
            return carry

        lax.fori_loop(0, NPW // 16, fbody, 0)
        pltpu.sync_copy(stg[comp].at[pl.ds(0, NPW)],
                        out_c[comp].at[pl.ds(sid * NPW, NPW)])


def kernel(h1, h2, coord1, coord2, edge_index, coord_diff, edge_attr, W0,
           b0, W1, b1, W2):
    del coord2
    n1 = edge_index[0].astype(jnp.int32)
    n2 = edge_index[1].astype(jnp.int32)
    w0a = W0[:H]
    w0b = W0[H:2 * H]
    w0c = W0[2 * H:].reshape(1, H)
    b0r = b0.reshape(1, H)
    b1r = b1.reshape(1, H)
    w2t = W2.reshape(1, H)

    n1r = jnp.pad(n1.reshape(NCHUNK, CH), ((0, NCHUNKP - NCHUNK), (0, 0)))
    n2r = jnp.pad(n2.reshape(NCHUNK, CH), ((0, NCHUNKP - NCHUNK), (0, 0)))
    a1, a2 = _precompute(h1, h2, w0a[:, 0::2], w0a[:, 1::2],
                         w0b[:, 0::2], w0b[:, 1::2])
    g1, g2 = _gather_rows(n1r, n2r, a1, a2)
    w1b = W1.astype(jnp.bfloat16)
    w1p = jnp.concatenate([w1b[0::2], w1b[1::2]], axis=0)
    txp, typ, tzp = _mlp(g1, g2, edge_attr, coord_diff,
                         w0c[:, 0::2], w0c[:, 1::2],
                         b0r[:, 0::2], b0r[:, 1::2],
                         w1p, b1r, w2t)
    c1p = jnp.pad(coord1, ((0, NPAD - N), (0, 0)))
    epad = (NCHUNKP - NCHUNK) * CH
    ox, oy, oz = _scatter_combine(
        n1r, jnp.pad(txp.reshape(E), (0, epad)),
        jnp.pad(typ.reshape(E), (0, epad)),
        jnp.pad(tzp.reshape(E), (0, epad)),
        c1p[:, 0], c1p[:, 1], c1p[:, 2])
    return jnp.stack([ox, oy, oz], axis=1)[:N]


# speedup vs baseline: 2.2568x; 1.3466x over previous
import functools

import jax
import jax.numpy as jnp
from jax import lax
from jax.experimental import pallas as pl
from jax.experimental.pallas import tpu as pltpu
from jax.experimental.pallas import tpu_sc as plsc

N = 10000
E = 160000
H = 256
NORM = 0.01

NPAD = 10240
CH = 128
NCHUNK = E // CH
NBLK = 1000
EBLK = 1280
NW_G = 32
NW_S = 16
NPW = NPAD // NW_S


HW = H // 2


def _pack_bf16_pair(even_f32, odd_f32):
    ie = lax.bitcast_convert_type(
        even_f32.astype(jnp.bfloat16).astype(jnp.float32), jnp.int32)
    io = lax.bitcast_convert_type(
        odd_f32.astype(jnp.bfloat16).astype(jnp.float32), jnp.int32)
    return io | lax.shift_right_logical(ie, 16)


def _precompute_body(h1_ref, h2_ref, w0ae_ref, w0ao_ref, w0be_ref,
                     w0bo_ref, a1_ref, a2_ref):
    a1_ref[...] = _pack_bf16_pair(
        jnp.dot(h1_ref[...], w0ae_ref[...],
                preferred_element_type=jnp.float32),
        jnp.dot(h1_ref[...], w0ao_ref[...],
                preferred_element_type=jnp.float32))
    a2_ref[...] = _pack_bf16_pair(
        jnp.dot(h2_ref[...], w0be_ref[...],
                preferred_element_type=jnp.float32),
        jnp.dot(h2_ref[...], w0bo_ref[...],
                preferred_element_type=jnp.float32))


_precompute = pl.pallas_call(
    _precompute_body,
    grid=(N // NBLK,),
    in_specs=[
        pl.BlockSpec((NBLK, H), lambda i: (i, 0)),
        pl.BlockSpec((NBLK, H), lambda i: (i, 0)),
        pl.BlockSpec((H, HW), lambda i: (0, 0)),
        pl.BlockSpec((H, HW), lambda i: (0, 0)),
        pl.BlockSpec((H, HW), lambda i: (0, 0)),
        pl.BlockSpec((H, HW), lambda i: (0, 0)),
    ],
    out_specs=[
        pl.BlockSpec((NBLK, HW), lambda i: (i, 0)),
        pl.BlockSpec((NBLK, HW), lambda i: (i, 0)),
    ],
    out_shape=[
        jax.ShapeDtypeStruct((N, HW), jnp.int32),
        jax.ShapeDtypeStruct((N, HW), jnp.int32),
    ],
)


_MESH_G = plsc.VectorSubcoreMesh(core_axis_name="c", subcore_axis_name="s")


MAXG = 40


@functools.partial(
    pl.kernel,
    out_type=(jax.ShapeDtypeStruct((E, HW), jnp.int32),
              jax.ShapeDtypeStruct((E, HW), jnp.int32)),
    mesh=_MESH_G,
    scratch_types=[
        pltpu.VMEM((MAXG, CH), jnp.int32),
        pltpu.VMEM((MAXG, CH), jnp.int32),
        pltpu.VMEM((CH, HW), jnp.int32),
        pltpu.VMEM((CH, HW), jnp.int32),
        pltpu.VMEM((CH, HW), jnp.int32),
        pltpu.VMEM((CH, HW), jnp.int32),
        pltpu.SemaphoreType.DMA,
        pltpu.SemaphoreType.DMA,
    ],
)
def _gather_rows(n1r, n2r, a1, a2, g1, g2, idx1_2d, idx2_2d, r1a, r1b,
                 r2a, r2b, semL, semG):
    wid = lax.axis_index("s") * 2 + lax.axis_index("c")
    rows1 = (r1a, r1b)
    rows2 = (r2a, r2b)
    start = pl.multiple_of(wid * MAXG, 8)
    nchunks = jnp.minimum(MAXG, NCHUNK - wid * MAXG)
    cpi1 = pltpu.async_copy(n1r.at[pl.ds(start, MAXG)], idx1_2d, semL)
    cpi2 = pltpu.async_copy(n2r.at[pl.ds(start, MAXG)], idx2_2d, semL)
    cpi1.wait()
    cpi2.wait()

    def fire(j, s):
        pltpu.async_copy(a1.at[idx1_2d.at[j]], rows1[s], semG)
        pltpu.async_copy(a2.at[idx2_2d.at[j]], rows2[s], semG)

    def wait_pair(s):
        pltpu.make_async_copy(a1.at[pl.ds(0, CH)], rows1[s], semG).wait()
        pltpu.make_async_copy(a2.at[pl.ds(0, CH)], rows2[s], semG).wait()

    fire(0, 0)

    def body(ip, carry):
        for s in (0, 1):
            j = 2 * ip + s

            @pl.when(j + 1 < nchunks)
            def _():
                fire(j + 1, 1 - s)

            wait_pair(s)
            base = pl.multiple_of((wid * MAXG + j) * CH, CH)
            pltpu.sync_copy(rows1[s], g1.at[pl.ds(base, CH)])
            pltpu.sync_copy(rows2[s], g2.at[pl.ds(base, CH)])
        return carry

    lax.fori_loop(0, nchunks // 2, body, 0)


def _mlp_body(g1, g2, ea, cdx, cdy, cdz, w0cl, w0ch, b0l, b0h, w1, b1,
              w2t, outx, outy, outz):
    i1 = g1[...]
    i2 = g2[...]
    hi = jnp.int32(-65536)
    e1 = lax.bitcast_convert_type(i1 << 16, jnp.float32)
    o1 = lax.bitcast_convert_type(i1 & hi, jnp.float32)
    e2 = lax.bitcast_convert_type(i2 << 16, jnp.float32)
    o2 = lax.bitcast_convert_type(i2 & hi, jnp.float32)
    eac = ea[...]
    eav = jnp.concatenate(
        [jnp.transpose(eac[0, c].reshape(1, CH))
         for c in range(EBLK // CH)], axis=0)
    pre_l = e1 + e2 + eav * w0cl[...] + b0l[...]
    pre_h = o1 + o2 + eav * w0ch[...] + b0h[...]
    x0l = (pre_l * jax.nn.sigmoid(pre_l)).astype(jnp.bfloat16)
    x0h = (pre_h * jax.nn.sigmoid(pre_h)).astype(jnp.bfloat16)
    x0 = jnp.concatenate([x0l, x0h], axis=1)
    pre1 = (jnp.dot(x0, w1[...], preferred_element_type=jnp.float32)
            + b1[...])
    x1 = pre1 * jax.nn.sigmoid(pre1)
    m = jnp.sum(x1 * w2t[...], axis=1, keepdims=True)
    mr = jnp.concatenate(
        [jnp.transpose(m[c * CH:(c + 1) * CH]) for c in range(EBLK // CH)],
        axis=0)
    outx[...] = (cdx[...].reshape(EBLK // CH, CH) * mr).reshape(
        1, EBLK // CH, CH)
    outy[...] = (cdy[...].reshape(EBLK // CH, CH) * mr).reshape(
        1, EBLK // CH, CH)
    outz[...] = (cdz[...].reshape(EBLK // CH, CH) * mr).reshape(
        1, EBLK // CH, CH)


_CHUNK3D = pl.BlockSpec((1, EBLK // CH, CH), lambda i: (i, 0, 0))

_mlp = pl.pallas_call(
    _mlp_body,
    grid=(E // EBLK,),
    in_specs=[
        pl.BlockSpec((EBLK, HW), lambda i: (i, 0)),
        pl.BlockSpec((EBLK, HW), lambda i: (i, 0)),
        _CHUNK3D,
        _CHUNK3D,
        _CHUNK3D,
        _CHUNK3D,
        pl.BlockSpec((1, HW), lambda i: (0, 0)),
        pl.BlockSpec((1, HW), lambda i: (0, 0)),
        pl.BlockSpec((1, HW), lambda i: (0, 0)),
        pl.BlockSpec((1, HW), lambda i: (0, 0)),
        pl.BlockSpec((H, H), lambda i: (0, 0)),
        pl.BlockSpec((1, H), lambda i: (0, 0)),
        pl.BlockSpec((1, H), lambda i: (0, 0)),
    ],
    out_specs=[_CHUNK3D] * 3,
    out_shape=[jax.ShapeDtypeStruct((E // EBLK, EBLK // CH, CH),
                                    jnp.float32)] * 3,
)


_MESH_S = plsc.VectorSubcoreMesh(core_axis_name="c", subcore_axis_name="s",
                                 num_cores=1)


MAXC = 80
NCHUNKP = MAXC * NW_S
DRAIN = 8


@functools.partial(
    pl.kernel,
    out_type=tuple(jax.ShapeDtypeStruct((NPAD,), jnp.float32)
                   for _ in range(3)),
    mesh=_MESH_S,
    scratch_types=[
        pltpu.VMEM((MAXC, CH), jnp.int32),
        pltpu.VMEM((MAXC * CH,), jnp.float32),
        pltpu.VMEM((MAXC * CH,), jnp.float32),
        pltpu.VMEM((MAXC * CH,), jnp.float32),
        pltpu.VMEM((NPW,), jnp.float32),
        pltpu.VMEM((CH * 3,), jnp.float32),
        pltpu.VMEM_SHARED((NPAD,), jnp.float32),
        pltpu.VMEM_SHARED((NPAD,), jnp.float32),
        pltpu.VMEM_SHARED((NPAD,), jnp.float32),
        pltpu.SemaphoreType.DMA,
        pltpu.SemaphoreType.DMA,
    ],
)
def _scatter_combine(n1r, tx, ty, tz, c1x, c1y, c1z, ox, oy, oz, idx2d,
                     stx, sty, stz, zbuf, dummy_v, acc_x, acc_y,
                     acc_z, semL, semS):
    sid = lax.axis_index("s")
    accs = (acc_x, acc_y, acc_z)
    stg = (stx, sty, stz)
    t_c = (tx, ty, tz)
    c1_c = (c1x, c1y, c1z)
    out_c = (ox, oy, oz)

    start = pl.multiple_of(sid * MAXC, 8)
    nchunks = jnp.minimum(MAXC, NCHUNK - sid * MAXC)
    cps = [pltpu.async_copy(n1r.at[pl.ds(start, MAXC)], idx2d, semL)]
    for comp in range(3):
        cps.append(pltpu.async_copy(
            t_c[comp].at[pl.ds(start * CH, MAXC * CH)], stg[comp], semL))

    def zbody(i, carry):
        zbuf[pl.ds(pl.multiple_of(i * 16, 16), 16)] = jnp.zeros(
            (16,), jnp.float32)
        return carry

    lax.fori_loop(0, NPW // 16, zbody, 0)
    for comp in range(3):
        pltpu.sync_copy(zbuf, accs[comp].at[pl.ds(sid * NPW, NPW)])
    plsc.subcore_barrier()
    for cp in cps:
        cp.wait()

    def fire(j, carry):
        sbase = pl.multiple_of(j * CH, CH)
        for comp in range(3):
            pltpu.async_copy(stg[comp].at[pl.ds(sbase, CH)],
                             accs[comp].at[idx2d.at[j]], semS, add=True)

        @pl.when(j >= DRAIN)
        def _():
            pltpu.make_async_copy(tx.at[pl.ds(0, CH * 3)], dummy_v,
                                  semS).wait()

        return carry

    lax.fori_loop(0, nchunks, fire, 0)

    def drain(j, carry):
        pltpu.make_async_copy(tx.at[pl.ds(0, CH * 3)], dummy_v,
                              semS).wait()
        return carry

    lax.fori_loop(0, DRAIN, drain, 0)
    plsc.subcore_barrier()

    for comp in range(3):
        pltpu.sync_copy(accs[comp].at[pl.ds(sid * NPW, NPW)],
                        stg[comp].at[pl.ds(0, NPW)])
        pltpu.sync_copy(c1_c[comp].at[pl.ds(sid * NPW, NPW)], zbuf)

        def fbody(i, carry):
            sl = pl.ds(pl.multiple_of(i * 16, 16), 16)
            stg[comp][sl] = zbuf[sl] + stg[comp][sl] * jnp.float32(NORM)
            return carry

        lax.fori_loop(0, NPW // 16, fbody, 0)
        pltpu.sync_copy(stg[comp].at[pl.ds(0, NPW)],
                        out_c[comp].at[pl.ds(sid * NPW, NPW)])


def kernel(h1, h2, coord1, coord2, edge_index, coord_diff, edge_attr, W0,
           b0, W1, b1, W2):
    del coord2
    n1 = edge_index[0].astype(jnp.int32)
    n2 = edge_index[1].astype(jnp.int32)
    w0a = W0[:H]
    w0b = W0[H:2 * H]
    w0c = W0[2 * H:].reshape(1, H)
    b0r = b0.reshape(1, H)
    b1r = b1.reshape(1, H)
    w2t = W2.reshape(1, H)

    n1r = jnp.pad(n1.reshape(NCHUNK, CH), ((0, NCHUNKP - NCHUNK), (0, 0)))
    n2r = jnp.pad(n2.reshape(NCHUNK, CH), ((0, NCHUNKP - NCHUNK), (0, 0)))
    a1, a2 = _precompute(h1, h2, w0a[:, :HW], w0a[:, HW:],
                         w0b[:, :HW], w0b[:, HW:])
    g1, g2 = _gather_rows(n1r, n2r, a1, a2)
    ea3 = edge_attr.reshape(E // EBLK, EBLK // CH, CH)
    cd3 = coord_diff.T.reshape(3, E // EBLK, EBLK // CH, CH)
    txp, typ, tzp = _mlp(g1, g2, ea3, cd3[0], cd3[1], cd3[2],
                         w0c[:, :HW], w0c[:, HW:],
                         b0r[:, :HW], b0r[:, HW:],
                         W1.astype(jnp.bfloat16), b1r, w2t)
    c1p = jnp.pad(coord1, ((0, NPAD - N), (0, 0)))
    epad = (NCHUNKP - NCHUNK) * CH
    ox, oy, oz = _scatter_combine(
        n1r, jnp.pad(txp.reshape(E), (0, epad)),
        jnp.pad(typ.reshape(E), (0, epad)),
        jnp.pad(tzp.reshape(E), (0, epad)),
        c1p[:, 0], c1p[:, 1], c1p[:, 2])
    return jnp.stack([ox, oy, oz], axis=1)[:N]


# speedup vs baseline: 2.4478x; 1.0846x over previous
import functools

import jax
import jax.numpy as jnp
from jax import lax
from jax.experimental import pallas as pl
from jax.experimental.pallas import tpu as pltpu
from jax.experimental.pallas import tpu_sc as plsc

N = 10000
E = 160000
H = 256
NORM = 0.01

NPAD = 10240
CH = 128
NCHUNK = E // CH
NBLK = 1000
EBLK = 1280
NW_G = 32
NW_S = 16
NPW = NPAD // NW_S


HW = H // 2


def _pack_bf16_pair(even_f32, odd_f32):
    ie = lax.bitcast_convert_type(
        even_f32.astype(jnp.bfloat16).astype(jnp.float32), jnp.int32)
    io = lax.bitcast_convert_type(
        odd_f32.astype(jnp.bfloat16).astype(jnp.float32), jnp.int32)
    return io | lax.shift_right_logical(ie, 16)


def _precompute_body(h1_ref, h2_ref, w0ae_ref, w0ao_ref, w0be_ref,
                     w0bo_ref, a1_ref, a2_ref):
    a1_ref[...] = _pack_bf16_pair(
        jnp.dot(h1_ref[...], w0ae_ref[...],
                preferred_element_type=jnp.float32),
        jnp.dot(h1_ref[...], w0ao_ref[...],
                preferred_element_type=jnp.float32))
    a2_ref[...] = _pack_bf16_pair(
        jnp.dot(h2_ref[...], w0be_ref[...],
                preferred_element_type=jnp.float32),
        jnp.dot(h2_ref[...], w0bo_ref[...],
                preferred_element_type=jnp.float32))


_precompute = pl.pallas_call(
    _precompute_body,
    grid=(N // NBLK,),
    in_specs=[
        pl.BlockSpec((NBLK, H), lambda i: (i, 0)),
        pl.BlockSpec((NBLK, H), lambda i: (i, 0)),
        pl.BlockSpec((H, HW), lambda i: (0, 0)),
        pl.BlockSpec((H, HW), lambda i: (0, 0)),
        pl.BlockSpec((H, HW), lambda i: (0, 0)),
        pl.BlockSpec((H, HW), lambda i: (0, 0)),
    ],
    out_specs=[
        pl.BlockSpec((NBLK, HW), lambda i: (i, 0)),
        pl.BlockSpec((NBLK, HW), lambda i: (i, 0)),
    ],
    out_shape=[
        jax.ShapeDtypeStruct((N, HW), jnp.int32),
        jax.ShapeDtypeStruct((N, HW), jnp.int32),
    ],
)


_MESH_G = plsc.VectorSubcoreMesh(core_axis_name="c", subcore_axis_name="s")


def _make_gather(c0, nch):
    cpw = -(-nch // NW_G)
    assert cpw % 2 == 0

    @functools.partial(
        pl.kernel,
        out_type=(jax.ShapeDtypeStruct((nch * CH, HW), jnp.int32),
                  jax.ShapeDtypeStruct((nch * CH, HW), jnp.int32)),
        mesh=_MESH_G,
        scratch_types=[
            pltpu.VMEM((cpw * CH,), jnp.int32),
            pltpu.VMEM((cpw * CH,), jnp.int32),
            pltpu.VMEM((CH, HW), jnp.int32),
            pltpu.VMEM((CH, HW), jnp.int32),
            pltpu.VMEM((CH, HW), jnp.int32),
            pltpu.VMEM((CH, HW), jnp.int32),
            pltpu.SemaphoreType.DMA,
            pltpu.SemaphoreType.DMA,
        ],
    )
    def gather(n1p, n2p, a1, a2, g1, g2, idx1f, idx2f, r1a, r1b, r2a, r2b,
               semL, semG):
        wid = lax.axis_index("s") * 2 + lax.axis_index("c")
        rows1 = (r1a, r1b)
        rows2 = (r2a, r2b)
        wchunk = wid * cpw
        nchunks = jnp.clip(nch - wchunk, 0, cpw)
        estart = pl.multiple_of((c0 + wchunk) * CH, CH)
        cpi1 = pltpu.async_copy(n1p.at[pl.ds(estart, cpw * CH)], idx1f,
                                semL)
        cpi2 = pltpu.async_copy(n2p.at[pl.ds(estart, cpw * CH)], idx2f,
                                semL)
        cpi1.wait()
        cpi2.wait()

        def fire(j, s):
            off = pl.multiple_of(j * CH, CH)
            pltpu.async_copy(a1.at[idx1f.at[pl.ds(off, CH)]], rows1[s],
                             semG)
            pltpu.async_copy(a2.at[idx2f.at[pl.ds(off, CH)]], rows2[s],
                             semG)

        def wait_pair(s):
            pltpu.make_async_copy(a1.at[pl.ds(0, CH)], rows1[s],
                                  semG).wait()
            pltpu.make_async_copy(a2.at[pl.ds(0, CH)], rows2[s],
                                  semG).wait()

        @pl.when(nchunks > 0)
        def _():
            fire(0, 0)

        def body(ip, carry):
            for s in (0, 1):
                j = 2 * ip + s

                @pl.when(j + 1 < nchunks)
                def _():
                    fire(j + 1, 1 - s)

                wait_pair(s)
                base = pl.multiple_of((wchunk + j) * CH, CH)
                pltpu.sync_copy(rows1[s], g1.at[pl.ds(base, CH)])
                pltpu.sync_copy(rows2[s], g2.at[pl.ds(base, CH)])
            return carry

        lax.fori_loop(0, nchunks // 2, body, 0)

    return gather


SPLIT_CH = 640
_gather_a = _make_gather(0, SPLIT_CH)
_gather_b = _make_gather(SPLIT_CH, NCHUNK - SPLIT_CH)


def _mlp_body(g1, g2, ea, cdx, cdy, cdz, w0cl, w0ch, b0l, b0h, w1, b1,
              w2t, outx, outy, outz):
    i1 = g1[...]
    i2 = g2[...]
    hi = jnp.int32(-65536)
    e1 = lax.bitcast_convert_type(i1 << 16, jnp.float32)
    o1 = lax.bitcast_convert_type(i1 & hi, jnp.float32)
    e2 = lax.bitcast_convert_type(i2 << 16, jnp.float32)
    o2 = lax.bitcast_convert_type(i2 & hi, jnp.float32)
    eac = ea[...]
    eav = jnp.concatenate(
        [jnp.transpose(eac[0, c].reshape(1, CH))
         for c in range(EBLK // CH)], axis=0)
    pre_l = e1 + e2 + eav * w0cl[...] + b0l[...]
    pre_h = o1 + o2 + eav * w0ch[...] + b0h[...]
    x0l = (pre_l * jax.nn.sigmoid(pre_l)).astype(jnp.bfloat16)
    x0h = (pre_h * jax.nn.sigmoid(pre_h)).astype(jnp.bfloat16)
    x0 = jnp.concatenate([x0l, x0h], axis=1)
    pre1 = (jnp.dot(x0, w1[...], preferred_element_type=jnp.float32)
            + b1[...])
    x1 = pre1 * jax.nn.sigmoid(pre1)
    m = jnp.sum(x1 * w2t[...], axis=1, keepdims=True)
    mr = jnp.concatenate(
        [jnp.transpose(m[c * CH:(c + 1) * CH]) for c in range(EBLK // CH)],
        axis=0)
    outx[...] = (cdx[...].reshape(EBLK // CH, CH) * mr).reshape(
        1, EBLK // CH, CH)
    outy[...] = (cdy[...].reshape(EBLK // CH, CH) * mr).reshape(
        1, EBLK // CH, CH)
    outz[...] = (cdz[...].reshape(EBLK // CH, CH) * mr).reshape(
        1, EBLK // CH, CH)


def _make_mlp(b0, nb):
    chunk_in = pl.BlockSpec((1, EBLK // CH, CH), lambda i: (b0 + i, 0, 0))
    chunk_out = pl.BlockSpec((1, EBLK // CH, CH), lambda i: (i, 0, 0))
    return pl.pallas_call(
        _mlp_body,
        grid=(nb,),
        in_specs=[
            pl.BlockSpec((EBLK, HW), lambda i: (i, 0)),
            pl.BlockSpec((EBLK, HW), lambda i: (i, 0)),
            chunk_in,
            chunk_in,
            chunk_in,
            chunk_in,
            pl.BlockSpec((1, HW), lambda i: (0, 0)),
            pl.BlockSpec((1, HW), lambda i: (0, 0)),
            pl.BlockSpec((1, HW), lambda i: (0, 0)),
            pl.BlockSpec((1, HW), lambda i: (0, 0)),
            pl.BlockSpec((H, H), lambda i: (0, 0)),
            pl.BlockSpec((1, H), lambda i: (0, 0)),
            pl.BlockSpec((1, H), lambda i: (0, 0)),
        ],
        out_specs=[chunk_out] * 3,
        out_shape=[jax.ShapeDtypeStruct((nb, EBLK // CH, CH),
                                        jnp.float32)] * 3,
    )


_BPH_A = SPLIT_CH * CH // EBLK
_BPH_B = (NCHUNK - SPLIT_CH) * CH // EBLK
_mlp_a = _make_mlp(0, _BPH_A)
_mlp_b = _make_mlp(_BPH_A, _BPH_B)


_MESH_S = plsc.VectorSubcoreMesh(core_axis_name="c", subcore_axis_name="s",
                                 num_cores=1)


MAXC = 80
NCHUNKP = MAXC * NW_S
DRAIN = 8


@functools.partial(
    pl.kernel,
    out_type=tuple(jax.ShapeDtypeStruct((NPAD,), jnp.float32)
                   for _ in range(3)),
    mesh=_MESH_S,
    scratch_types=[
        pltpu.VMEM((MAXC, CH), jnp.int32),
        pltpu.VMEM((MAXC * CH,), jnp.float32),
        pltpu.VMEM((MAXC * CH,), jnp.float32),
        pltpu.VMEM((MAXC * CH,), jnp.float32),
        pltpu.VMEM((NPW,), jnp.float32),
        pltpu.VMEM((CH * 3,), jnp.float32),
        pltpu.VMEM_SHARED((NPAD,), jnp.float32),
        pltpu.VMEM_SHARED((NPAD,), jnp.float32),
        pltpu.VMEM_SHARED((NPAD,), jnp.float32),
        pltpu.SemaphoreType.DMA,
        pltpu.SemaphoreType.DMA,
    ],
)
def _scatter_combine(n1r, tx, ty, tz, c1x, c1y, c1z, ox, oy, oz, idx2d,
                     stx, sty, stz, zbuf, dummy_v, acc_x, acc_y,
                     acc_z, semL, semS):
    sid = lax.axis_index("s")
    accs = (acc_x, acc_y, acc_z)
    stg = (stx, sty, stz)
    t_c = (tx, ty, tz)
    c1_c = (c1x, c1y, c1z)
    out_c = (ox, oy, oz)

    start = pl.multiple_of(sid * MAXC, 8)
    nchunks = jnp.minimum(MAXC, NCHUNK - sid * MAXC)
    cps = [pltpu.async_copy(n1r.at[pl.ds(start, MAXC)], idx2d, semL)]
    for comp in range(3):
        cps.append(pltpu.async_copy(
            t_c[comp].at[pl.ds(start * CH, MAXC * CH)], stg[comp], semL))

    def zbody(i, carry):
        zbuf[pl.ds(pl.multiple_of(i * 16, 16), 16)] = jnp.zeros(
            (16,), jnp.float32)
        return carry

    lax.fori_loop(0, NPW // 16, zbody, 0)
    for comp in range(3):
        pltpu.sync_copy(zbuf, accs[comp].at[pl.ds(sid * NPW, NPW)])
    plsc.subcore_barrier()
    for cp in cps:
        cp.wait()

    def fire(j, carry):
        sbase = pl.multiple_of(j * CH, CH)
        for comp in range(3):
            pltpu.async_copy(stg[comp].at[pl.ds(sbase, CH)],
                             accs[comp].at[idx2d.at[j]], semS, add=True)

        @pl.when(j >= DRAIN)
        def _():
            pltpu.make_async_copy(tx.at[pl.ds(0, CH * 3)], dummy_v,
                                  semS).wait()

        return carry

    lax.fori_loop(0, nchunks, fire, 0)

    def drain(j, carry):
        pltpu.make_async_copy(tx.at[pl.ds(0, CH * 3)], dummy_v,
                              semS).wait()
        return carry

    lax.fori_loop(0, DRAIN, drain, 0)
    plsc.subcore_barrier()

    for comp in range(3):
        pltpu.sync_copy(accs[comp].at[pl.ds(sid * NPW, NPW)],
                        stg[comp].at[pl.ds(0, NPW)])
        pltpu.sync_copy(c1_c[comp].at[pl.ds(sid * NPW, NPW)], zbuf)

        def fbody(i, carry):
            sl = pl.ds(pl.multiple_of(i * 16, 16), 16)
            stg[comp][sl] = zbuf[sl] + stg[comp][sl] * jnp.float32(NORM)
            return carry

        lax.fori_loop(0, NPW // 16, fbody, 0)
        pltpu.sync_copy(stg[comp].at[pl.ds(0, NPW)],
                        out_c[comp].at[pl.ds(sid * NPW, NPW)])


def kernel(h1, h2, coord1, coord2, edge_index, coord_diff, edge_attr, W0,
           b0, W1, b1, W2):
    del coord2
    n1 = edge_index[0].astype(jnp.int32)
    n2 = edge_index[1].astype(jnp.int32)
    w0a = W0[:H]
    w0b = W0[H:2 * H]
    w0c = W0[2 * H:].reshape(1, H)
    b0r = b0.reshape(1, H)
    b1r = b1.reshape(1, H)
    w2t = W2.reshape(1, H)

    n1r = jnp.pad(n1.reshape(NCHUNK, CH), ((0, NCHUNKP - NCHUNK), (0, 0)))
    n2r = jnp.pad(n2.reshape(NCHUNK, CH), ((0, NCHUNKP - NCHUNK), (0, 0)))
    a1, a2 = _precompute(h1, h2, w0a[:, :HW], w0a[:, HW:],
                         w0b[:, :HW], w0b[:, HW:])
    n1p = n1r.reshape(NCHUNKP * CH)
    n2p = n2r.reshape(NCHUNKP * CH)
    ga1, ga2 = _gather_a(n1p, n2p, a1, a2)
    gb1, gb2 = _gather_b(n1p, n2p, a1, a2)
    ea3 = edge_attr.reshape(E // EBLK, EBLK // CH, CH)
    cd3 = coord_diff.T.reshape(3, E // EBLK, EBLK // CH, CH)
    w1b = W1.astype(jnp.bfloat16)
    consts = (w0c[:, :HW], w0c[:, HW:], b0r[:, :HW], b0r[:, HW:],
              w1b, b1r, w2t)
    ta = _mlp_a(ga1, ga2, ea3, cd3[0], cd3[1], cd3[2], *consts)
    tb = _mlp_b(gb1, gb2, ea3, cd3[0], cd3[1], cd3[2], *consts)
    txp, typ, tzp = (jnp.concatenate([a_, b_], axis=0)
                     for a_, b_ in zip(ta, tb))
    c1p = jnp.pad(coord1, ((0, NPAD - N), (0, 0)))
    epad = (NCHUNKP - NCHUNK) * CH
    ox, oy, oz = _scatter_combine(
        n1r, jnp.pad(txp.reshape(E), (0, epad)),
        jnp.pad(typ.reshape(E), (0, epad)),
        jnp.pad(tzp.reshape(E), (0, epad)),
        c1p[:, 0], c1p[:, 1], c1p[:, 2])
    return jnp.stack([ox, oy, oz], axis=1)[:N]


# speedup vs baseline: 2.4767x; 1.0118x over previous
import functools

import jax
import jax.numpy as jnp
from jax import lax
from jax.experimental import pallas as pl
from jax.experimental.pallas import tpu as pltpu
from jax.experimental.pallas import tpu_sc as plsc

N = 10000
E = 160000
H = 256
NORM = 0.01

NPAD = 10240
CH = 128
NCHUNK = E // CH
NBLK = 1000
EBLK = 1280
NW_G = 32
NW_S = 16
NPW = NPAD // NW_S


HW = H // 2


def _pack_bf16_pair(even_f32, odd_f32):
    ie = lax.bitcast_convert_type(
        even_f32.astype(jnp.bfloat16).astype(jnp.float32), jnp.int32)
    io = lax.bitcast_convert_type(
        odd_f32.astype(jnp.bfloat16).astype(jnp.float32), jnp.int32)
    return io | lax.shift_right_logical(ie, 16)


def _precompute_body(h1_ref, h2_ref, w0ae_ref, w0ao_ref, w0be_ref,
                     w0bo_ref, a1_ref, a2_ref):
    a1_ref[...] = _pack_bf16_pair(
        jnp.dot(h1_ref[...], w0ae_ref[...],
                preferred_element_type=jnp.float32),
        jnp.dot(h1_ref[...], w0ao_ref[...],
                preferred_element_type=jnp.float32))
    a2_ref[...] = _pack_bf16_pair(
        jnp.dot(h2_ref[...], w0be_ref[...],
                preferred_element_type=jnp.float32),
        jnp.dot(h2_ref[...], w0bo_ref[...],
                preferred_element_type=jnp.float32))


_precompute = pl.pallas_call(
    _precompute_body,
    grid=(N // NBLK,),
    in_specs=[
        pl.BlockSpec((NBLK, H), lambda i: (i, 0)),
        pl.BlockSpec((NBLK, H), lambda i: (i, 0)),
        pl.BlockSpec((H, HW), lambda i: (0, 0)),
        pl.BlockSpec((H, HW), lambda i: (0, 0)),
        pl.BlockSpec((H, HW), lambda i: (0, 0)),
        pl.BlockSpec((H, HW), lambda i: (0, 0)),
    ],
    out_specs=[
        pl.BlockSpec((NBLK, HW), lambda i: (i, 0)),
        pl.BlockSpec((NBLK, HW), lambda i: (i, 0)),
    ],
    out_shape=[
        jax.ShapeDtypeStruct((N, HW), jnp.int32),
        jax.ShapeDtypeStruct((N, HW), jnp.int32),
    ],
)


_MESH_G = plsc.VectorSubcoreMesh(core_axis_name="c", subcore_axis_name="s")


def _make_gather(c0, nch):
    cpw = -(-nch // NW_G)
    assert cpw % 2 == 0

    @functools.partial(
        pl.kernel,
        out_type=(jax.ShapeDtypeStruct((nch * CH, HW), jnp.int32),
                  jax.ShapeDtypeStruct((nch * CH, HW), jnp.int32)),
        mesh=_MESH_G,
        scratch_types=[
            pltpu.VMEM((cpw * CH,), jnp.int32),
            pltpu.VMEM((cpw * CH,), jnp.int32),
            pltpu.VMEM((CH, HW), jnp.int32),
            pltpu.VMEM((CH, HW), jnp.int32),
            pltpu.VMEM((CH, HW), jnp.int32),
            pltpu.VMEM((CH, HW), jnp.int32),
            pltpu.SemaphoreType.DMA,
            pltpu.SemaphoreType.DMA,
        ],
    )
    def gather(n1p, n2p, a1, a2, g1, g2, idx1f, idx2f, r1a, r1b, r2a, r2b,
               semL, semG):
        wid = lax.axis_index("s") * 2 + lax.axis_index("c")
        rows1 = (r1a, r1b)
        rows2 = (r2a, r2b)
        wchunk = wid * cpw
        nchunks = jnp.clip(nch - wchunk, 0, cpw)
        estart = pl.multiple_of((c0 + wchunk) * CH, CH)
        cpi1 = pltpu.async_copy(n1p.at[pl.ds(estart, cpw * CH)], idx1f,
                                semL)
        cpi2 = pltpu.async_copy(n2p.at[pl.ds(estart, cpw * CH)], idx2f,
                                semL)
        cpi1.wait()
        cpi2.wait()

        def fire(j, s):
            off = pl.multiple_of(j * CH, CH)
            pltpu.async_copy(a1.at[idx1f.at[pl.ds(off, CH)]], rows1[s],
                             semG)
            pltpu.async_copy(a2.at[idx2f.at[pl.ds(off, CH)]], rows2[s],
                             semG)

        def wait_pair(s):
            pltpu.make_async_copy(a1.at[pl.ds(0, CH)], rows1[s],
                                  semG).wait()
            pltpu.make_async_copy(a2.at[pl.ds(0, CH)], rows2[s],
                                  semG).wait()

        @pl.when(nchunks > 0)
        def _():
            fire(0, 0)

        def body(ip, carry):
            for s in (0, 1):
                j = 2 * ip + s

                @pl.when(j + 1 < nchunks)
                def _():
                    fire(j + 1, 1 - s)

                wait_pair(s)
                base = pl.multiple_of((wchunk + j) * CH, CH)
                pltpu.sync_copy(rows1[s], g1.at[pl.ds(base, CH)])
                pltpu.sync_copy(rows2[s], g2.at[pl.ds(base, CH)])
            return carry

        lax.fori_loop(0, nchunks // 2, body, 0)

    return gather


SPLIT_CH = 510
_gather_a = _make_gather(0, SPLIT_CH)
_gather_b = _make_gather(SPLIT_CH, NCHUNK - SPLIT_CH)


def _mlp_body(g1, g2, ea, cdx, cdy, cdz, w0cl, w0ch, b0l, b0h, w1, b1,
              w2t, outx, outy, outz):
    i1 = g1[...]
    i2 = g2[...]
    hi = jnp.int32(-65536)
    e1 = lax.bitcast_convert_type(i1 << 16, jnp.float32)
    o1 = lax.bitcast_convert_type(i1 & hi, jnp.float32)
    e2 = lax.bitcast_convert_type(i2 << 16, jnp.float32)
    o2 = lax.bitcast_convert_type(i2 & hi, jnp.float32)
    eac = ea[...]
    eav = jnp.concatenate(
        [jnp.transpose(eac[0, c].reshape(1, CH))
         for c in range(EBLK // CH)], axis=0)
    pre_l = e1 + e2 + eav * w0cl[...] + b0l[...]
    pre_h = o1 + o2 + eav * w0ch[...] + b0h[...]
    x0l = (pre_l * jax.nn.sigmoid(pre_l)).astype(jnp.bfloat16)
    x0h = (pre_h * jax.nn.sigmoid(pre_h)).astype(jnp.bfloat16)
    x0 = jnp.concatenate([x0l, x0h], axis=1)
    pre1 = (jnp.dot(x0, w1[...], preferred_element_type=jnp.float32)
            + b1[...])
    x1 = pre1 * jax.nn.sigmoid(pre1)
    m = jnp.sum(x1 * w2t[...], axis=1, keepdims=True)
    mr = jnp.concatenate(
        [jnp.transpose(m[c * CH:(c + 1) * CH]) for c in range(EBLK // CH)],
        axis=0)
    outx[...] = (cdx[...].reshape(EBLK // CH, CH) * mr).reshape(
        1, EBLK // CH, CH)
    outy[...] = (cdy[...].reshape(EBLK // CH, CH) * mr).reshape(
        1, EBLK // CH, CH)
    outz[...] = (cdz[...].reshape(EBLK // CH, CH) * mr).reshape(
        1, EBLK // CH, CH)


def _make_mlp(b0, nb):
    chunk_in = pl.BlockSpec((1, EBLK // CH, CH), lambda i: (b0 + i, 0, 0))
    chunk_out = pl.BlockSpec((1, EBLK // CH, CH), lambda i: (i, 0, 0))
    return pl.pallas_call(
        _mlp_body,
        grid=(nb,),
        in_specs=[
            pl.BlockSpec((EBLK, HW), lambda i: (i, 0)),
            pl.BlockSpec((EBLK, HW), lambda i: (i, 0)),
            chunk_in,
            chunk_in,
            chunk_in,
            chunk_in,
            pl.BlockSpec((1, HW), lambda i: (0, 0)),
            pl.BlockSpec((1, HW), lambda i: (0, 0)),
            pl.BlockSpec((1, HW), lambda i: (0, 0)),
            pl.BlockSpec((1, HW), lambda i: (0, 0)),
            pl.BlockSpec((H, H), lambda i: (0, 0)),
            pl.BlockSpec((1, H), lambda i: (0, 0)),
            pl.BlockSpec((1, H), lambda i: (0, 0)),
        ],
        out_specs=[chunk_out] * 3,
        out_shape=[jax.ShapeDtypeStruct((nb, EBLK // CH, CH),
                                        jnp.float32)] * 3,
    )


_BPH_A = SPLIT_CH * CH // EBLK
_BPH_B = (NCHUNK - SPLIT_CH) * CH // EBLK
_mlp_a = _make_mlp(0, _BPH_A)
_mlp_b = _make_mlp(_BPH_A, _BPH_B)


_MESH_S = plsc.VectorSubcoreMesh(core_axis_name="c", subcore_axis_name="s",
                                 num_cores=1)


MAXC = 80
NCHUNKP = MAXC * NW_S
DRAIN = 8


@functools.partial(
    pl.kernel,
    out_type=tuple(jax.ShapeDtypeStruct((NPAD,), jnp.float32)
                   for _ in range(3)),
    mesh=_MESH_S,
    scratch_types=[
        pltpu.VMEM((MAXC, CH), jnp.int32),
        pltpu.VMEM((MAXC * CH,), jnp.float32),
        pltpu.VMEM((MAXC * CH,), jnp.float32),
        pltpu.VMEM((MAXC * CH,), jnp.float32),
        pltpu.VMEM((NPW,), jnp.float32),
        pltpu.VMEM((CH * 3,), jnp.float32),
        pltpu.VMEM_SHARED((NPAD,), jnp.float32),
        pltpu.VMEM_SHARED((NPAD,), jnp.float32),
        pltpu.VMEM_SHARED((NPAD,), jnp.float32),
        pltpu.SemaphoreType.DMA,
        pltpu.SemaphoreType.DMA,
    ],
)
def _scatter_combine(n1r, tx, ty, tz, c1x, c1y, c1z, ox, oy, oz, idx2d,
                     stx, sty, stz, zbuf, dummy_v, acc_x, acc_y,
                     acc_z, semL, semS):
    sid = lax.axis_index("s")
    accs = (acc_x, acc_y, acc_z)
    stg = (stx, sty, stz)
    t_c = (tx, ty, tz)
    c1_c = (c1x, c1y, c1z)
    out_c = (ox, oy, oz)

    start = pl.multiple_of(sid * MAXC, 8)
    nchunks = jnp.minimum(MAXC, NCHUNK - sid * MAXC)
    cps = [pltpu.async_copy(n1r.at[pl.ds(start, MAXC)], idx2d, semL)]
    for comp in range(3):
        cps.append(pltpu.async_copy(
            t_c[comp].at[pl.ds(start * CH, MAXC * CH)], stg[comp], semL))

    def zbody(i, carry):
        zbuf[pl.ds(pl.multiple_of(i * 16, 16), 16)] = jnp.zeros(
            (16,), jnp.float32)
        return carry

    lax.fori_loop(0, NPW // 16, zbody, 0)
    for comp in range(3):
        pltpu.sync_copy(zbuf, accs[comp].at[pl.ds(sid * NPW, NPW)])
    plsc.subcore_barrier()
    for cp in cps:
        cp.wait()

    def fire(j, carry):
        sbase = pl.multiple_of(j * CH, CH)
        for comp in range(3):
            pltpu.async_copy(stg[comp].at[pl.ds(sbase, CH)],
                             accs[comp].at[idx2d.at[j]], semS, add=True)

        @pl.when(j >= DRAIN)
        def _():
            pltpu.make_async_copy(tx.at[pl.ds(0, CH * 3)], dummy_v,
                                  semS).wait()

        return carry

    lax.fori_loop(0, nchunks, fire, 0)

    def drain(j, carry):
        pltpu.make_async_copy(tx.at[pl.ds(0, CH * 3)], dummy_v,
                              semS).wait()
        return carry

    lax.fori_loop(0, DRAIN, drain, 0)
    plsc.subcore_barrier()

    for comp in range(3):
        pltpu.sync_copy(accs[comp].at[pl.ds(sid * NPW, NPW)],
                        stg[comp].at[pl.ds(0, NPW)])
        pltpu.sync_copy(c1_c[comp].at[pl.ds(sid * NPW, NPW)], zbuf)

        def fbody(i, carry):
            sl = pl.ds(pl.multiple_of(i * 16, 16), 16)
            stg[comp][sl] = zbuf[sl] + stg[comp][sl] * jnp.float32(NORM)
            return carry

        lax.fori_loop(0, NPW // 16, fbody, 0)
        pltpu.sync_copy(stg[comp].at[pl.ds(0, NPW)],
                        out_c[comp].at[pl.ds(sid * NPW, NPW)])


def kernel(h1, h2, coord1, coord2, edge_index, coord_diff, edge_attr, W0,
           b0, W1, b1, W2):
    del coord2
    n1 = edge_index[0].astype(jnp.int32)
    n2 = edge_index[1].astype(jnp.int32)
    w0a = W0[:H]
    w0b = W0[H:2 * H]
    w0c = W0[2 * H:].reshape(1, H)
    b0r = b0.reshape(1, H)
    b1r = b1.reshape(1, H)
    w2t = W2.reshape(1, H)

    n1r = jnp.pad(n1.reshape(NCHUNK, CH), ((0, NCHUNKP - NCHUNK), (0, 0)))
    n2r = jnp.pad(n2.reshape(NCHUNK, CH), ((0, NCHUNKP - NCHUNK), (0, 0)))
    a1, a2 = _precompute(h1, h2, w0a[:, :HW], w0a[:, HW:],
                         w0b[:, :HW], w0b[:, HW:])
    n1p = n1r.reshape(NCHUNKP * CH)
    n2p = n2r.reshape(NCHUNKP * CH)
    ga1, ga2 = _gather_a(n1p, n2p, a1, a2)
    gb1, gb2 = _gather_b(n1p, n2p, a1, a2)
    ea3 = edge_attr.reshape(E // EBLK, EBLK // CH, CH)
    cd3 = coord_diff.T.reshape(3, E // EBLK, EBLK // CH, CH)
    w1b = W1.astype(jnp.bfloat16)
    consts = (w0c[:, :HW], w0c[:, HW:], b0r[:, :HW], b0r[:, HW:],
              w1b, b1r, w2t)
    ta = _mlp_a(ga1, ga2, ea3, cd3[0], cd3[1], cd3[2], *consts)
    tb = _mlp_b(gb1, gb2, ea3, cd3[0], cd3[1], cd3[2], *consts)
    txp, typ, tzp = (jnp.concatenate([a_, b_], axis=0)
                     for a_, b_ in zip(ta, tb))
    c1p = jnp.pad(coord1, ((0, NPAD - N), (0, 0)))
    epad = (NCHUNKP - NCHUNK) * CH
    ox, oy, oz = _scatter_combine(
        n1r, jnp.pad(txp.reshape(E), (0, epad)),
        jnp.pad(typ.reshape(E), (0, epad)),
        jnp.pad(tzp.reshape(E), (0, epad)),
        c1p[:, 0], c1p[:, 1], c1p[:, 2])
    return jnp.stack([ox, oy, oz], axis=1)[:N]


# speedup vs baseline: 2.6853x; 1.0842x over previous
import functools

import jax
import jax.numpy as jnp
from jax import lax
from jax.experimental import pallas as pl
from jax.experimental.pallas import tpu as pltpu
from jax.experimental.pallas import tpu_sc as plsc

N = 10000
E = 160000
H = 256
NORM = 0.01

NPAD = 10240
CH = 128
NCHUNK = E // CH
NBLK = 1000
EBLK = 3200
NW_G = 32
NW_S = 16
NPW = NPAD // NW_S


HW = H // 2


def _pack_bf16_pair(even_f32, odd_f32):
    ie = lax.bitcast_convert_type(
        even_f32.astype(jnp.bfloat16).astype(jnp.float32), jnp.int32)
    io = lax.bitcast_convert_type(
        odd_f32.astype(jnp.bfloat16).astype(jnp.float32), jnp.int32)
    return io | lax.shift_right_logical(ie, 16)


def _precompute_body(h1_ref, h2_ref, w0ae_ref, w0ao_ref, w0be_ref,
                     w0bo_ref, a1_ref, a2_ref):
    a1_ref[...] = _pack_bf16_pair(
        jnp.dot(h1_ref[...], w0ae_ref[...],
                preferred_element_type=jnp.float32),
        jnp.dot(h1_ref[...], w0ao_ref[...],
                preferred_element_type=jnp.float32))
    a2_ref[...] = _pack_bf16_pair(
        jnp.dot(h2_ref[...], w0be_ref[...],
                preferred_element_type=jnp.float32),
        jnp.dot(h2_ref[...], w0bo_ref[...],
                preferred_element_type=jnp.float32))


_precompute = pl.pallas_call(
    _precompute_body,
    grid=(N // NBLK,),
    in_specs=[
        pl.BlockSpec((NBLK, H), lambda i: (i, 0)),
        pl.BlockSpec((NBLK, H), lambda i: (i, 0)),
        pl.BlockSpec((H, HW), lambda i: (0, 0)),
        pl.BlockSpec((H, HW), lambda i: (0, 0)),
        pl.BlockSpec((H, HW), lambda i: (0, 0)),
        pl.BlockSpec((H, HW), lambda i: (0, 0)),
    ],
    out_specs=[
        pl.BlockSpec((NBLK, HW), lambda i: (i, 0)),
        pl.BlockSpec((NBLK, HW), lambda i: (i, 0)),
    ],
    out_shape=[
        jax.ShapeDtypeStruct((N, HW), jnp.int32),
        jax.ShapeDtypeStruct((N, HW), jnp.int32),
    ],
)


_MESH_G = plsc.VectorSubcoreMesh(core_axis_name="c", subcore_axis_name="s")


def _make_gather(c0, nch):
    cpw = -(-nch // NW_G)
    assert cpw % 2 == 0

    @functools.partial(
        pl.kernel,
        out_type=(jax.ShapeDtypeStruct((nch * CH, HW), jnp.int32),
                  jax.ShapeDtypeStruct((nch * CH, HW), jnp.int32)),
        mesh=_MESH_G,
        scratch_types=[
            pltpu.VMEM((cpw * CH,), jnp.int32),
            pltpu.VMEM((cpw * CH,), jnp.int32),
            pltpu.VMEM((CH, HW), jnp.int32),
            pltpu.VMEM((CH, HW), jnp.int32),
            pltpu.VMEM((CH, HW), jnp.int32),
            pltpu.VMEM((CH, HW), jnp.int32),
            pltpu.SemaphoreType.DMA,
            pltpu.SemaphoreType.DMA,
        ],
    )
    def gather(n1p, n2p, a1, a2, g1, g2, idx1f, idx2f, r1a, r1b, r2a, r2b,
               semL, semG):
        wid = lax.axis_index("s") * 2 + lax.axis_index("c")
        rows1 = (r1a, r1b)
        rows2 = (r2a, r2b)
        wchunk = wid * cpw
        nchunks = jnp.clip(nch - wchunk, 0, cpw)
        estart = pl.multiple_of((c0 + wchunk) * CH, CH)
        cpi1 = pltpu.async_copy(n1p.at[pl.ds(estart, cpw * CH)], idx1f,
                                semL)
        cpi2 = pltpu.async_copy(n2p.at[pl.ds(estart, cpw * CH)], idx2f,
                                semL)
        cpi1.wait()
        cpi2.wait()

        def fire(j, s):
            off = pl.multiple_of(j * CH, CH)
            pltpu.async_copy(a1.at[idx1f.at[pl.ds(off, CH)]], rows1[s],
                             semG)
            pltpu.async_copy(a2.at[idx2f.at[pl.ds(off, CH)]], rows2[s],
                             semG)

        def wait_pair(s):
            pltpu.make_async_copy(a1.at[pl.ds(0, CH)], rows1[s],
                                  semG).wait()
            pltpu.make_async_copy(a2.at[pl.ds(0, CH)], rows2[s],
                                  semG).wait()

        @pl.when(nchunks > 0)
        def _():
            fire(0, 0)

        def body(ip, carry):
            for s in (0, 1):
                j = 2 * ip + s

                @pl.when(j + 1 < nchunks)
                def _():
                    fire(j + 1, 1 - s)

                wait_pair(s)
                base = pl.multiple_of((wchunk + j) * CH, CH)
                pltpu.sync_copy(rows1[s], g1.at[pl.ds(base, CH)])
                pltpu.sync_copy(rows2[s], g2.at[pl.ds(base, CH)])
            return carry

        lax.fori_loop(0, nchunks // 2, body, 0)

    return gather


SPLIT_CH = 500
_gather_a = _make_gather(0, SPLIT_CH)
_gather_b = _make_gather(SPLIT_CH, NCHUNK - SPLIT_CH)


def _mlp_body(g1, g2, ea, cdx, cdy, cdz, w0cl, w0ch, b0l, b0h, w1, b1,
              w2t, outx, outy, outz):
    i1 = g1[...]
    i2 = g2[...]
    hi = jnp.int32(-65536)
    e1 = lax.bitcast_convert_type(i1 << 16, jnp.float32)
    o1 = lax.bitcast_convert_type(i1 & hi, jnp.float32)
    e2 = lax.bitcast_convert_type(i2 << 16, jnp.float32)
    o2 = lax.bitcast_convert_type(i2 & hi, jnp.float32)
    eac = ea[...]
    eav = jnp.concatenate(
        [jnp.transpose(eac[0, c].reshape(1, CH))
         for c in range(EBLK // CH)], axis=0)
    pre_l = e1 + e2 + eav * w0cl[...] + b0l[...]
    pre_h = o1 + o2 + eav * w0ch[...] + b0h[...]
    x0l = (pre_l * jax.nn.sigmoid(pre_l)).astype(jnp.bfloat16)
    x0h = (pre_h * jax.nn.sigmoid(pre_h)).astype(jnp.bfloat16)
    x0 = jnp.concatenate([x0l, x0h], axis=1)
    pre1 = (jnp.dot(x0, w1[...], preferred_element_type=jnp.float32)
            + b1[...])
    x1 = pre1 * jax.nn.sigmoid(pre1)
    m = jnp.sum(x1 * w2t[...], axis=1, keepdims=True)
    mr = jnp.concatenate(
        [jnp.transpose(m[c * CH:(c + 1) * CH]) for c in range(EBLK // CH)],
        axis=0)
    outx[...] = (cdx[...].reshape(EBLK // CH, CH) * mr).reshape(
        1, EBLK // CH, CH)
    outy[...] = (cdy[...].reshape(EBLK // CH, CH) * mr).reshape(
        1, EBLK // CH, CH)
    outz[...] = (cdz[...].reshape(EBLK // CH, CH) * mr).reshape(
        1, EBLK // CH, CH)


def _make_mlp(b0, nb):
    chunk_in = pl.BlockSpec((1, EBLK // CH, CH), lambda i: (b0 + i, 0, 0))
    chunk_out = pl.BlockSpec((1, EBLK // CH, CH), lambda i: (i, 0, 0))
    return pl.pallas_call(
        _mlp_body,
        grid=(nb,),
        in_specs=[
            pl.BlockSpec((EBLK, HW), lambda i: (i, 0)),
            pl.BlockSpec((EBLK, HW), lambda i: (i, 0)),
            chunk_in,
            chunk_in,
            chunk_in,
            chunk_in,
            pl.BlockSpec((1, HW), lambda i: (0, 0)),
            pl.BlockSpec((1, HW), lambda i: (0, 0)),
            pl.BlockSpec((1, HW), lambda i: (0, 0)),
            pl.BlockSpec((1, HW), lambda i: (0, 0)),
            pl.BlockSpec((H, H), lambda i: (0, 0)),
            pl.BlockSpec((1, H), lambda i: (0, 0)),
            pl.BlockSpec((1, H), lambda i: (0, 0)),
        ],
        out_specs=[chunk_out] * 3,
        out_shape=[jax.ShapeDtypeStruct((nb, EBLK // CH, CH),
                                        jnp.float32)] * 3,
    )


_BPH_A = SPLIT_CH * CH // EBLK
_BPH_B = (NCHUNK - SPLIT_CH) * CH // EBLK
_mlp_a = _make_mlp(0, _BPH_A)
_mlp_b = _make_mlp(_BPH_A, _BPH_B)


_MESH_S = plsc.VectorSubcoreMesh(core_axis_name="c", subcore_axis_name="s",
                                 num_cores=1)


MAXC = 80
NCHUNKP = MAXC * NW_S
DRAIN = 8


@functools.partial(
    pl.kernel,
    out_type=tuple(jax.ShapeDtypeStruct((NPAD,), jnp.float32)
                   for _ in range(3)),
    mesh=_MESH_S,
    scratch_types=[
        pltpu.VMEM((MAXC, CH), jnp.int32),
        pltpu.VMEM((MAXC * CH,), jnp.float32),
        pltpu.VMEM((MAXC * CH,), jnp.float32),
        pltpu.VMEM((MAXC * CH,), jnp.float32),
        pltpu.VMEM((NPW,), jnp.float32),
        pltpu.VMEM((CH * 3,), jnp.float32),
        pltpu.VMEM_SHARED((NPAD,), jnp.float32),
        pltpu.VMEM_SHARED((NPAD,), jnp.float32),
        pltpu.VMEM_SHARED((NPAD,), jnp.float32),
        pltpu.SemaphoreType.DMA,
        pltpu.SemaphoreType.DMA,
    ],
)
def _scatter_combine(n1r, tx, ty, tz, c1x, c1y, c1z, ox, oy, oz, idx2d,
                     stx, sty, stz, zbuf, dummy_v, acc_x, acc_y,
                     acc_z, semL, semS):
    sid = lax.axis_index("s")
    accs = (acc_x, acc_y, acc_z)
    stg = (stx, sty, stz)
    t_c = (tx, ty, tz)
    c1_c = (c1x, c1y, c1z)
    out_c = (ox, oy, oz)

    start = pl.multiple_of(sid * MAXC, 8)
    nchunks = jnp.minimum(MAXC, NCHUNK - sid * MAXC)
    cps = [pltpu.async_copy(n1r.at[pl.ds(start, MAXC)], idx2d, semL)]
    for comp in range(3):
        cps.append(pltpu.async_copy(
            t_c[comp].at[pl.ds(start * CH, MAXC * CH)], stg[comp], semL))

    def zbody(i, carry):
        zbuf[pl.ds(pl.multiple_of(i * 16, 16), 16)] = jnp.zeros(
            (16,), jnp.float32)
        return carry

    lax.fori_loop(0, NPW // 16, zbody, 0)
    for comp in range(3):
        pltpu.sync_copy(zbuf, accs[comp].at[pl.ds(sid * NPW, NPW)])
    plsc.subcore_barrier()
    for cp in cps:
        cp.wait()

    def fire(j, carry):
        sbase = pl.multiple_of(j * CH, CH)
        for comp in range(3):
            pltpu.async_copy(stg[comp].at[pl.ds(sbase, CH)],
                             accs[comp].at[idx2d.at[j]], semS, add=True)

        @pl.when(j >= DRAIN)
        def _():
            pltpu.make_async_copy(tx.at[pl.ds(0, CH * 3)], dummy_v,
                                  semS).wait()

        return carry

    lax.fori_loop(0, nchunks, fire, 0)

    def drain(j, carry):
        pltpu.make_async_copy(tx.at[pl.ds(0, CH * 3)], dummy_v,
                              semS).wait()
        return carry

    lax.fori_loop(0, DRAIN, drain, 0)
    plsc.subcore_barrier()

    for comp in range(3):
        pltpu.sync_copy(accs[comp].at[pl.ds(sid * NPW, NPW)],
                        stg[comp].at[pl.ds(0, NPW)])
        pltpu.sync_copy(c1_c[comp].at[pl.ds(sid * NPW, NPW)], zbuf)

        def fbody(i, carry):
            sl = pl.ds(pl.multiple_of(i * 16, 16), 16)
            stg[comp][sl] = zbuf[sl] + stg[comp][sl] * jnp.float32(NORM)
            return carry

        lax.fori_loop(0, NPW // 16, fbody, 0)
        pltpu.sync_copy(stg[comp].at[pl.ds(0, NPW)],
                        out_c[comp].at[pl.ds(sid * NPW, NPW)])


def kernel(h1, h2, coord1, coord2, edge_index, coord_diff, edge_attr, W0,
           b0, W1, b1, W2):
    del coord2
    n1 = edge_index[0].astype(jnp.int32)
    n2 = edge_index[1].astype(jnp.int32)
    w0a = W0[:H]
    w0b = W0[H:2 * H]
    w0c = W0[2 * H:].reshape(1, H)
    b0r = b0.reshape(1, H)
    b1r = b1.reshape(1, H)
    w2t = W2.reshape(1, H)

    n1r = jnp.pad(n1.reshape(NCHUNK, CH), ((0, NCHUNKP - NCHUNK), (0, 0)))
    n2r = jnp.pad(n2.reshape(NCHUNK, CH), ((0, NCHUNKP - NCHUNK), (0, 0)))
    a1, a2 = _precompute(h1, h2, w0a[:, :HW], w0a[:, HW:],
                         w0b[:, :HW], w0b[:, HW:])
    n1p = n1r.reshape(NCHUNKP * CH)
    n2p = n2r.reshape(NCHUNKP * CH)
    ga1, ga2 = _gather_a(n1p, n2p, a1, a2)
    gb1, gb2 = _gather_b(n1p, n2p, a1, a2)
    ea3 = edge_attr.reshape(E // EBLK, EBLK // CH, CH)
    cd3 = coord_diff.T.reshape(3, E // EBLK, EBLK // CH, CH)
    w1b = W1.astype(jnp.bfloat16)
    consts = (w0c[:, :HW], w0c[:, HW:], b0r[:, :HW], b0r[:, HW:],
              w1b, b1r, w2t)
    ta = _mlp_a(ga1, ga2, ea3, cd3[0], cd3[1], cd3[2], *consts)
    tb = _mlp_b(gb1, gb2, ea3, cd3[0], cd3[1], cd3[2], *consts)
    txp, typ, tzp = (jnp.concatenate([a_, b_], axis=0)
                     for a_, b_ in zip(ta, tb))
    c1p = jnp.pad(coord1, ((0, NPAD - N), (0, 0)))
    epad = (NCHUNKP - NCHUNK) * CH
    ox, oy, oz = _scatter_combine(
        n1r, jnp.pad(txp.reshape(E), (0, epad)),
        jnp.pad(typ.reshape(E), (0, epad)),
        jnp.pad(tzp.reshape(E), (0, epad)),
        c1p[:, 0], c1p[:, 1], c1p[:, 2])
    return jnp.stack([ox, oy, oz], axis=1)[:N]
